# baseline (speedup 1.0000x reference)
import functools

import jax
import jax.numpy as jnp
from jax import lax
from jax.experimental import pallas as pl
from jax.experimental.pallas import tpu as pltpu
from jax.experimental.pallas import tpu_sc as plsc

BATCH = 16384
_INFO = plsc.get_sparse_core_info()
_NC, _NS, _L = _INFO.num_cores, _INFO.num_subcores, _INFO.num_lanes
_NW = _NC * _NS
_BPW = BATCH // _NW


def _mean_model_sc(uid_hbm, mid_hbm, utab_hbm, mtab_hbm, mean_hbm, out_hbm,
                   uid_v, mid_v, ub_v, mb_v, mean_v, sem_u, sem_m):
    wid = lax.axis_index("s") * _NC + lax.axis_index("c")
    base = wid * _BPW

    ci_u = pltpu.async_copy(uid_hbm.at[pl.ds(base, _BPW)], uid_v, sem_u)
    ci_m = pltpu.async_copy(mid_hbm.at[pl.ds(base, _BPW)], mid_v, sem_m)
    pltpu.sync_copy(mean_hbm, mean_v)
    ci_u.wait()
    cu = pltpu.async_copy(utab_hbm.at[uid_v], ub_v, sem_u)
    ci_m.wait()
    cm = pltpu.async_copy(mtab_hbm.at[mid_v], mb_v, sem_m)
    cu.wait()
    cm.wait()

    mean_vec = mean_v[...]
    for i in range(_BPW // _L):
        sl = pl.ds(i * _L, _L)
        ub_v[sl] = ub_v[sl] + mb_v[sl] + mean_vec

    pltpu.sync_copy(ub_v, out_hbm.at[pl.ds(base, _BPW)])


@jax.jit
def _run(uid, mid, utab, mtab, mean_vec):
    mesh = plsc.VectorSubcoreMesh(core_axis_name="c", subcore_axis_name="s")
    k = functools.partial(
        pl.kernel,
        mesh=mesh,
        out_type=jax.ShapeDtypeStruct((BATCH,), jnp.float32),
        scratch_types=[
            pltpu.VMEM((_BPW,), jnp.int32),
            pltpu.VMEM((_BPW,), jnp.int32),
            pltpu.VMEM((_BPW,), jnp.float32),
            pltpu.VMEM((_BPW,), jnp.float32),
            pltpu.VMEM((_L,), jnp.float32),
            pltpu.SemaphoreType.DMA,
            pltpu.SemaphoreType.DMA,
        ],
    )(_mean_model_sc)
    return k(uid, mid, utab, mtab, mean_vec)


def kernel(userId, movieId, user_table, movie_table, mean):
    uid = userId.astype(jnp.int32)
    mid = movieId.astype(jnp.int32)
    mean_vec = jnp.broadcast_to(jnp.asarray(mean, jnp.float32), (_L,))
    return _run(uid, mid, user_table, movie_table, mean_vec)


# speedup vs baseline: 1.0046x; 1.0046x over previous
import functools

import jax
import jax.numpy as jnp
from jax import lax
from jax.experimental import pallas as pl
from jax.experimental.pallas import tpu as pltpu
from jax.experimental.pallas import tpu_sc as plsc

BATCH = 16384
_INFO = plsc.get_sparse_core_info()
_NC, _NS, _L = _INFO.num_cores, _INFO.num_subcores, _INFO.num_lanes
_NW = _NC * _NS
_BPW = BATCH // _NW


def _mean_model_sc(uid_hbm, mid_hbm, utab_hbm, mtab_hbm, mean_hbm, out_hbm,
                   uid_v, mid_v, ub_v, mb_v, mean_v, sem_u, sem_m):
    wid = lax.axis_index("s") * _NC + lax.axis_index("c")
    base = wid * _BPW
    half = _BPW // 2

    ci_u = pltpu.async_copy(uid_hbm.at[pl.ds(base, _BPW)], uid_v, sem_u)
    ci_m = pltpu.async_copy(mid_hbm.at[pl.ds(base, _BPW)], mid_v, sem_m)
    pltpu.sync_copy(mean_hbm, mean_v)
    ci_u.wait()
    cu = pltpu.async_copy(utab_hbm.at[uid_v], ub_v, sem_u)
    ci_m.wait()
    cm = pltpu.async_copy(mtab_hbm.at[mid_v], mb_v, sem_m)
    cu.wait()
    cm.wait()

    mean_vec = mean_v[...]
    for i in range(half // _L):
        sl = pl.ds(i * _L, _L)
        ub_v[sl] = ub_v[sl] + mb_v[sl] + mean_vec
    cw = pltpu.async_copy(ub_v.at[pl.ds(0, half)],
                          out_hbm.at[pl.ds(base, half)], sem_u)
    for i in range(half // _L, _BPW // _L):
        sl = pl.ds(i * _L, _L)
        ub_v[sl] = ub_v[sl] + mb_v[sl] + mean_vec
    cw2 = pltpu.async_copy(ub_v.at[pl.ds(half, half)],
                           out_hbm.at[pl.ds(base + half, half)], sem_m)
    cw.wait()
    cw2.wait()


@jax.jit
def _run(uid, mid, utab, mtab, mean_vec):
    mesh = plsc.VectorSubcoreMesh(core_axis_name="c", subcore_axis_name="s")
    k = functools.partial(
        pl.kernel,
        mesh=mesh,
        out_type=jax.ShapeDtypeStruct((BATCH,), jnp.float32),
        scratch_types=[
            pltpu.VMEM((_BPW,), jnp.int32),
            pltpu.VMEM((_BPW,), jnp.int32),
            pltpu.VMEM((_BPW,), jnp.float32),
            pltpu.VMEM((_BPW,), jnp.float32),
            pltpu.VMEM((_L,), jnp.float32),
            pltpu.SemaphoreType.DMA,
            pltpu.SemaphoreType.DMA,
        ],
    )(_mean_model_sc)
    return k(uid, mid, utab, mtab, mean_vec)


def kernel(userId, movieId, user_table, movie_table, mean):
    uid = userId.astype(jnp.int32)
    mid = movieId.astype(jnp.int32)
    mean_vec = jnp.broadcast_to(jnp.asarray(mean, jnp.float32), (_L,))
    return _run(uid, mid, user_table, movie_table, mean_vec)
